# baseline (speedup 1.0000x reference)
import functools

import jax
import jax.numpy as jnp
from jax import lax
from jax.experimental import pallas as pl
from jax.experimental.pallas import tpu as pltpu
from jax.experimental.pallas import tpu_sc as plsc

NC = 2
NS = 16
NW = NC * NS

V = 1000000
BATCH = 4096
L = 50
LP = 64
D = 32
COLS = BATCH // NW
B_C = 8
N_CHUNK = COLS // B_C


def _make_lookup():
    mesh = plsc.VectorSubcoreMesh(core_axis_name="c", subcore_axis_name="s")

    @functools.partial(
        pl.kernel,
        mesh=mesh,
        compiler_params=pltpu.CompilerParams(use_tc_tiling_on_sc=True),
        out_type=jax.ShapeDtypeStruct((BATCH, L, D), jnp.float32),
        scratch_types=[
            pltpu.VMEM((B_C, LP), jnp.int32),
            pltpu.VMEM((LP, 8, D), jnp.float32),
            pltpu.VMEM((B_C, L, D), jnp.float32),
            pltpu.SemaphoreType.DMA,
        ],
    )
    def lookup(table_hbm, idx_hbm, out_hbm, idx_v, slab_v, out_v, sem):
        w = lax.axis_index("s") * NC + lax.axis_index("c")

        def chunk_body(c, carry):
            b0 = w * COLS + c * B_C
            pltpu.sync_copy(idx_hbm.at[pl.ds(b0, B_C)], idx_v)

            def row_body(bl, carry2):
                def fire_body(q, carry3):
                    vec = idx_v[bl, pl.ds(q * 16, 16)]
                    for k in range(16):
                        i = vec[k]
                        pltpu.make_async_copy(
                            table_hbm.at[i >> 3],
                            slab_v.at[q * 16 + k],
                            sem,
                        ).start()
                    return carry3

                def wait_body(q, carry3):
                    for _k in range(16):
                        pltpu.make_async_copy(
                            table_hbm.at[0], slab_v.at[0], sem
                        ).wait()
                    return carry3

                def sel_body(q, carry3):
                    vec = idx_v[bl, pl.ds(q * 16, 16)]
                    for k in range(16):
                        li = q * 16 + k

                        @pl.when(li < L)
                        def _():
                            r = vec[k] & 7
                            out_v[bl, li, pl.ds(0, 16)] = slab_v[
                                li, r, pl.ds(0, 16)
                            ]
                            out_v[bl, li, pl.ds(16, 16)] = slab_v[
                                li, r, pl.ds(16, 16)
                            ]

                    return carry3

                lax.fori_loop(0, 4, fire_body, 0)
                lax.fori_loop(0, 4, wait_body, 0)
                lax.fori_loop(0, 4, sel_body, 0)
                return carry2

            lax.fori_loop(0, B_C, row_body, 0)
            pltpu.sync_copy(out_v, out_hbm.at[pl.ds(b0, B_C)])
            return carry

        lax.fori_loop(0, N_CHUNK, chunk_body, 0)

    return lookup


_lookup = _make_lookup()


@jax.jit
def kernel(inputs, embeddings):
    pad = (
        jnp.arange(BATCH, dtype=jnp.int32)[:, None] * 31
        + jnp.arange(LP - L, dtype=jnp.int32)[None, :] * 65519
        + 12345
    ) % V
    idx = jnp.concatenate([inputs.astype(jnp.int32), pad], axis=1)
    return _lookup(embeddings.reshape(V // 8, 8, D), idx)


# speedup vs baseline: 1.2361x; 1.2361x over previous
import functools

import jax
import jax.numpy as jnp
from jax import lax
from jax.experimental import pallas as pl
from jax.experimental.pallas import tpu as pltpu
from jax.experimental.pallas import tpu_sc as plsc

NC = 2
NS = 16
NW = NC * NS

V = 1000000
BATCH = 4096
L = 50
D = 32
COLS = BATCH // NW
L_C = 10
N_CHUNK = L // L_C


def _make_lookup():
    mesh = plsc.VectorSubcoreMesh(core_axis_name="c", subcore_axis_name="s")

    @functools.partial(
        pl.kernel,
        mesh=mesh,
        compiler_params=pltpu.CompilerParams(use_tc_tiling_on_sc=False),
        out_type=jax.ShapeDtypeStruct((L, BATCH, D), jnp.float32),
        scratch_types=[
            pltpu.VMEM((L_C, COLS), jnp.int32),
            pltpu.VMEM((L_C, COLS, D), jnp.float32),
            pltpu.SemaphoreType.DMA,
        ],
    )
    def lookup(table_hbm, idx_hbm, out_hbm, idx_v, rows_v, sem):
        w = lax.axis_index("s") * NC + lax.axis_index("c")
        col0 = w * COLS

        def chunk_body(c, carry):
            l0 = c * L_C
            pltpu.sync_copy(
                idx_hbm.at[pl.ds(l0, L_C), pl.ds(col0, COLS)], idx_v
            )
            gathers = [
                pltpu.async_copy(
                    table_hbm.at[idx_v.at[li]], rows_v.at[li], sem
                )
                for li in range(L_C)
            ]
            for h in gathers:
                h.wait()
            pltpu.sync_copy(
                rows_v, out_hbm.at[pl.ds(l0, L_C), pl.ds(col0, COLS)]
            )
            return carry

        lax.fori_loop(0, N_CHUNK, chunk_body, 0)

    return lookup


_lookup = _make_lookup()


@jax.jit
def kernel(inputs, embeddings):
    tflat = lax.optimization_barrier(embeddings.reshape(-1))
    out3 = _lookup(tflat.reshape(V, D), inputs.T.astype(jnp.int32))
    return out3.transpose(1, 0, 2)
